# Initial kernel scaffold; baseline (speedup 1.0000x reference)
import jax
import jax.numpy as jnp
from jax.experimental import pallas as pl


def kernel(adj, x, w1, b1, w2, b2, wres, bres):
    raise NotImplementedError("write your pallas kernel here")


# speedup vs baseline: 1.2750x; 1.2750x over previous
import jax
import jax.numpy as jnp
from jax.experimental import pallas as pl
from jax.experimental.pallas import tpu as pltpu

_PARALLEL = pltpu.CompilerParams(dimension_semantics=("parallel",))


def _q_kernel(x_ref, w1_ref, wcat_ref, b1_ref, bcat_ref, q_ref):
    p = jnp.dot(x_ref[...], w1_ref[...], preferred_element_type=jnp.float32)
    q = jnp.dot(p, wcat_ref[...], preferred_element_type=jnp.float32)
    cc = (
        jnp.dot(b1_ref[...], wcat_ref[...], preferred_element_type=jnp.float32)
        + bcat_ref[...]
    )
    q_ref[...] = q + cc


def _agg_kernel(adj_ref, q_ref, o_ref):
    o_ref[...] = jnp.dot(
        adj_ref[...], q_ref[...], preferred_element_type=jnp.float32
    )


def _agg_relu_res_kernel(adj_ref, p2_ref, res_ref, o_ref):
    base = jnp.dot(adj_ref[...], p2_ref[...], preferred_element_type=jnp.float32)
    o_ref[...] = jnp.maximum(base, 0.0) + res_ref[...]


def _pick_tile(n, cap=512):
    if n <= cap:
        return n
    t = cap
    while t >= 8:
        if n % t == 0:
            return t
        t //= 2
    return n


def kernel(adj, x, w1, b1, w2, b2, wres, bres):
    n, nfeat = x.shape
    nhid = w1.shape[1]
    nclass = w2.shape[1]

    wcat = jnp.concatenate([w2, wres], axis=1)
    bcat = jnp.concatenate([b2, bres], axis=1)
    fcat = wcat.shape[1]

    tile = _pick_tile(n)
    grid = (n // tile,)

    q = pl.pallas_call(
        _q_kernel,
        grid=grid,
        in_specs=[
            pl.BlockSpec((tile, nfeat), lambda i: (i, 0)),
            pl.BlockSpec((nfeat, nhid), lambda i: (0, 0)),
            pl.BlockSpec((nhid, fcat), lambda i: (0, 0)),
            pl.BlockSpec((1, nhid), lambda i: (0, 0)),
            pl.BlockSpec((1, fcat), lambda i: (0, 0)),
        ],
        out_specs=pl.BlockSpec((tile, fcat), lambda i: (i, 0)),
        out_shape=jax.ShapeDtypeStruct((n, fcat), jnp.float32),
        compiler_params=_PARALLEL,
    )(x, w1, wcat, b1, bcat)

    r = pl.pallas_call(
        _agg_kernel,
        grid=grid,
        in_specs=[
            pl.BlockSpec((tile, n), lambda i: (i, 0)),
            pl.BlockSpec((n, fcat), lambda i: (0, 0)),
        ],
        out_specs=pl.BlockSpec((tile, fcat), lambda i: (i, 0)),
        out_shape=jax.ShapeDtypeStruct((n, fcat), jnp.float32),
        compiler_params=_PARALLEL,
    )(adj, q)

    out = pl.pallas_call(
        _agg_relu_res_kernel,
        grid=grid,
        in_specs=[
            pl.BlockSpec((tile, n), lambda i: (i, 0)),
            pl.BlockSpec((n, nclass), lambda i: (0, 0)),
            pl.BlockSpec((tile, nclass), lambda i: (i, 1)),
        ],
        out_specs=pl.BlockSpec((tile, nclass), lambda i: (i, 0)),
        out_shape=jax.ShapeDtypeStruct((n, nclass), jnp.float32),
        compiler_params=_PARALLEL,
    )(adj, r, r)

    return out
